# Initial kernel scaffold; baseline (speedup 1.0000x reference)
import jax
import jax.numpy as jnp
from jax.experimental import pallas as pl


def kernel(x, pos_table):
    raise NotImplementedError("write your pallas kernel here")


# speedup vs baseline: 1.6640x; 1.6640x over previous
import functools

import jax
import jax.numpy as jnp
from jax.experimental import pallas as pl
from jax.experimental.pallas import tpu as pltpu

_BS = 1024


def _add_kernel(x_ref, pos_ref, out_ref):
    out_ref[...] = x_ref[...] + pos_ref[...][None, :, :]


@functools.partial(jax.jit, static_argnames=())
def kernel(x, pos_table):
    batch, seq_len, d_model = x.shape
    pos = pos_table[:seq_len]
    grid = (seq_len // _BS, batch)
    return pl.pallas_call(
        _add_kernel,
        grid=grid,
        in_specs=[
            pl.BlockSpec((1, _BS, d_model), lambda s, b: (b, s, 0)),
            pl.BlockSpec((_BS, d_model), lambda s, b: (s, 0)),
        ],
        out_specs=pl.BlockSpec((1, _BS, d_model), lambda s, b: (b, s, 0)),
        out_shape=jax.ShapeDtypeStruct(x.shape, x.dtype),
        compiler_params=pltpu.CompilerParams(
            dimension_semantics=("arbitrary", "arbitrary"),
        ),
    )(x, pos)


# speedup vs baseline: 1.7389x; 1.0450x over previous
import functools

import jax
import jax.numpy as jnp
from jax.experimental import pallas as pl
from jax.experimental.pallas import tpu as pltpu

_BS = 2048


def _add_kernel(x_ref, pos_ref, out_ref):
    out_ref[...] = x_ref[...] + pos_ref[...][None, :, :]


@functools.partial(jax.jit, static_argnames=())
def kernel(x, pos_table):
    batch, seq_len, d_model = x.shape
    pos = pos_table[:seq_len]
    grid = (seq_len // _BS, batch)
    return pl.pallas_call(
        _add_kernel,
        grid=grid,
        in_specs=[
            pl.BlockSpec((1, _BS, d_model), lambda s, b: (b, s, 0)),
            pl.BlockSpec((_BS, d_model), lambda s, b: (s, 0)),
        ],
        out_specs=pl.BlockSpec((1, _BS, d_model), lambda s, b: (b, s, 0)),
        out_shape=jax.ShapeDtypeStruct(x.shape, x.dtype),
        compiler_params=pltpu.CompilerParams(
            dimension_semantics=("arbitrary", "arbitrary"),
        ),
    )(x, pos)
